# baseline (speedup 1.0000x reference)
import functools

import jax
import jax.numpy as jnp
from jax.experimental import pallas as pl

B, C, H, W = 8, 96, 384, 384
TH = 32
NH = H // TH
NL = 8


def _pool_stats_kernel(x_ref, m_ref, pooled_ref, part_ref):
    xb = x_ref[0]
    mx = jnp.max(xb, axis=0)
    mn = jnp.min(xb, axis=0)
    me = jnp.sum(xb, axis=0) * (1.0 / C)
    m = (m_ref[0] == 1).astype(jnp.float32)

    pooled_ref[0, 0] = mx
    pooled_ref[0, 1] = me
    pooled_ref[0, 2] = mn

    s1x = jnp.sum(mx * m)
    s1e = jnp.sum(me * m)
    s1n = jnp.sum(mn * m)
    s2x = jnp.sum(mx * mx * m)
    s2e = jnp.sum(me * me * m)
    s2n = jnp.sum(mn * mn * m)
    cnt = jnp.sum(m)
    part_ref[0, 0, 0, :] = jnp.stack(
        [s1x, s1e, s1n, s2x, s2e, s2n, cnt, cnt])


def _normalize_kernel(pooled_ref, m_ref, part_ref, out_ref):
    b = pl.program_id(0)
    s = jnp.sum(part_ref[b], axis=(0, 1))
    cnt = s[6]
    keep = m_ref[0] == 1
    for c in range(3):
        mean = s[c] / cnt
        var = (s[3 + c] - s[c] * s[c] / cnt) / (cnt - 1.0)
        rstd = jax.lax.rsqrt(var)
        out_ref[0, c] = jnp.where(
            keep, (pooled_ref[0, c] - mean) * rstd, 0.0)


@jax.jit
def kernel(x, mask):
    mask = mask.astype(jnp.int32)

    pooled, part = pl.pallas_call(
        _pool_stats_kernel,
        grid=(B, NH),
        in_specs=[
            pl.BlockSpec((1, C, TH, W), lambda b, h: (b, 0, h, 0)),
            pl.BlockSpec((1, TH, W), lambda b, h: (b, h, 0)),
        ],
        out_specs=[
            pl.BlockSpec((1, 3, TH, W), lambda b, h: (b, 0, h, 0)),
            pl.BlockSpec((1, 1, 1, NL), lambda b, h: (b, h, 0, 0)),
        ],
        out_shape=[
            jax.ShapeDtypeStruct((B, 3, H, W), jnp.float32),
            jax.ShapeDtypeStruct((B, NH, 1, NL), jnp.float32),
        ],
    )(x, mask)

    out = pl.pallas_call(
        _normalize_kernel,
        grid=(B, NH),
        in_specs=[
            pl.BlockSpec((1, 3, TH, W), lambda b, h: (b, 0, h, 0)),
            pl.BlockSpec((1, TH, W), lambda b, h: (b, h, 0)),
            pl.BlockSpec((B, NH, 1, NL), lambda b, h: (0, 0, 0, 0)),
        ],
        out_specs=pl.BlockSpec((1, 3, TH, W), lambda b, h: (b, 0, h, 0)),
        out_shape=jax.ShapeDtypeStruct((B, 3, H, W), jnp.float32),
    )(pooled, mask, part)

    return out


# speedup vs baseline: 1.2767x; 1.2767x over previous
import functools

import jax
import jax.numpy as jnp
from jax.experimental import pallas as pl

B, C, H, W = 8, 96, 384, 384
TH = 64
NH = H // TH
NL = 8


def _pool_stats_kernel(x_ref, m_ref, pooled_ref, part_ref):
    xb = x_ref[0]
    mx = jnp.max(xb, axis=0)
    mn = jnp.min(xb, axis=0)
    me = jnp.sum(xb, axis=0) * (1.0 / C)
    m = (m_ref[0] == 1).astype(jnp.float32)

    pooled_ref[0, 0] = mx
    pooled_ref[0, 1] = me
    pooled_ref[0, 2] = mn

    s1x = jnp.sum(mx * m)
    s1e = jnp.sum(me * m)
    s1n = jnp.sum(mn * m)
    s2x = jnp.sum(mx * mx * m)
    s2e = jnp.sum(me * me * m)
    s2n = jnp.sum(mn * mn * m)
    cnt = jnp.sum(m)
    part_ref[0, 0, 0, :] = jnp.stack(
        [s1x, s1e, s1n, s2x, s2e, s2n, cnt, cnt])


def _normalize_kernel(pooled_ref, m_ref, part_ref, out_ref):
    b = pl.program_id(0)
    s = jnp.sum(part_ref[b], axis=(0, 1))
    cnt = s[6]
    keep = m_ref[0] == 1
    for c in range(3):
        mean = s[c] / cnt
        var = (s[3 + c] - s[c] * s[c] / cnt) / (cnt - 1.0)
        rstd = jax.lax.rsqrt(var)
        out_ref[0, c] = jnp.where(
            keep, (pooled_ref[0, c] - mean) * rstd, 0.0)


@jax.jit
def kernel(x, mask):
    mask = mask.astype(jnp.int32)

    pooled, part = pl.pallas_call(
        _pool_stats_kernel,
        grid=(B, NH),
        in_specs=[
            pl.BlockSpec((1, C, TH, W), lambda b, h: (b, 0, h, 0)),
            pl.BlockSpec((1, TH, W), lambda b, h: (b, h, 0)),
        ],
        out_specs=[
            pl.BlockSpec((1, 3, TH, W), lambda b, h: (b, 0, h, 0)),
            pl.BlockSpec((1, 1, 1, NL), lambda b, h: (b, h, 0, 0)),
        ],
        out_shape=[
            jax.ShapeDtypeStruct((B, 3, H, W), jnp.float32),
            jax.ShapeDtypeStruct((B, NH, 1, NL), jnp.float32),
        ],
    )(x, mask)

    out = pl.pallas_call(
        _normalize_kernel,
        grid=(B, NH),
        in_specs=[
            pl.BlockSpec((1, 3, TH, W), lambda b, h: (b, 0, h, 0)),
            pl.BlockSpec((1, TH, W), lambda b, h: (b, h, 0)),
            pl.BlockSpec((B, NH, 1, NL), lambda b, h: (0, 0, 0, 0)),
        ],
        out_specs=pl.BlockSpec((1, 3, TH, W), lambda b, h: (b, 0, h, 0)),
        out_shape=jax.ShapeDtypeStruct((B, 3, H, W), jnp.float32),
    )(pooled, mask, part)

    return out


# speedup vs baseline: 1.3606x; 1.0657x over previous
import functools

import jax
import jax.numpy as jnp
from jax.experimental import pallas as pl

B, C, H, W = 8, 96, 384, 384
TH = 128
NH = H // TH
NL = 8


def _pool_stats_kernel(x_ref, m_ref, pooled_ref, part_ref):
    xb = x_ref[0]
    mx = jnp.max(xb, axis=0)
    mn = jnp.min(xb, axis=0)
    me = jnp.sum(xb, axis=0) * (1.0 / C)
    m = (m_ref[0] == 1).astype(jnp.float32)

    pooled_ref[0, 0] = mx
    pooled_ref[0, 1] = me
    pooled_ref[0, 2] = mn

    s1x = jnp.sum(mx * m)
    s1e = jnp.sum(me * m)
    s1n = jnp.sum(mn * m)
    s2x = jnp.sum(mx * mx * m)
    s2e = jnp.sum(me * me * m)
    s2n = jnp.sum(mn * mn * m)
    cnt = jnp.sum(m)
    part_ref[0, 0, 0, :] = jnp.stack(
        [s1x, s1e, s1n, s2x, s2e, s2n, cnt, cnt])


def _normalize_kernel(pooled_ref, m_ref, part_ref, out_ref):
    b = pl.program_id(0)
    s = jnp.sum(part_ref[b], axis=(0, 1))
    cnt = s[6]
    keep = m_ref[0] == 1
    for c in range(3):
        mean = s[c] / cnt
        var = (s[3 + c] - s[c] * s[c] / cnt) / (cnt - 1.0)
        rstd = jax.lax.rsqrt(var)
        out_ref[0, c] = jnp.where(
            keep, (pooled_ref[0, c] - mean) * rstd, 0.0)


@jax.jit
def kernel(x, mask):
    mask = mask.astype(jnp.int32)

    pooled, part = pl.pallas_call(
        _pool_stats_kernel,
        grid=(B, NH),
        in_specs=[
            pl.BlockSpec((1, C, TH, W), lambda b, h: (b, 0, h, 0)),
            pl.BlockSpec((1, TH, W), lambda b, h: (b, h, 0)),
        ],
        out_specs=[
            pl.BlockSpec((1, 3, TH, W), lambda b, h: (b, 0, h, 0)),
            pl.BlockSpec((1, 1, 1, NL), lambda b, h: (b, h, 0, 0)),
        ],
        out_shape=[
            jax.ShapeDtypeStruct((B, 3, H, W), jnp.float32),
            jax.ShapeDtypeStruct((B, NH, 1, NL), jnp.float32),
        ],
    )(x, mask)

    out = pl.pallas_call(
        _normalize_kernel,
        grid=(B, NH),
        in_specs=[
            pl.BlockSpec((1, 3, TH, W), lambda b, h: (b, 0, h, 0)),
            pl.BlockSpec((1, TH, W), lambda b, h: (b, h, 0)),
            pl.BlockSpec((B, NH, 1, NL), lambda b, h: (0, 0, 0, 0)),
        ],
        out_specs=pl.BlockSpec((1, 3, TH, W), lambda b, h: (b, 0, h, 0)),
        out_shape=jax.ShapeDtypeStruct((B, 3, H, W), jnp.float32),
    )(pooled, mask, part)

    return out


# speedup vs baseline: 1.4170x; 1.0414x over previous
import jax
import jax.numpy as jnp
from jax.experimental import pallas as pl
from jax.experimental.pallas import tpu as pltpu

B, C, H, W = 8, 96, 384, 384
CC = 16
NC = C // CC
OH = H // NC


def _fused_kernel(x_ref, m_ref, out_ref, pooled_s, maskf_s, stats_s):
    b = pl.program_id(0)
    k = pl.program_id(1)
    slot = jax.lax.rem(b, 2)

    @pl.when(b < B)
    def _pool():
        xb = x_ref[0]
        cmx = jnp.max(xb, axis=0)
        cmn = jnp.min(xb, axis=0)
        csm = jnp.sum(xb, axis=0)

        @pl.when(k == 0)
        def _():
            pooled_s[slot, 0] = cmx
            pooled_s[slot, 1] = csm
            pooled_s[slot, 2] = cmn
            maskf_s[slot] = (m_ref[0] == 1).astype(jnp.float32)

        @pl.when(k > 0)
        def _():
            pooled_s[slot, 0] = jnp.maximum(pooled_s[slot, 0], cmx)
            pooled_s[slot, 1] = pooled_s[slot, 1] + csm
            pooled_s[slot, 2] = jnp.minimum(pooled_s[slot, 2], cmn)

        @pl.when(k == NC - 1)
        def _():
            me = pooled_s[slot, 1] * (1.0 / C)
            pooled_s[slot, 1] = me
            mf = maskf_s[slot]
            mx = pooled_s[slot, 0]
            mn = pooled_s[slot, 2]
            stats_s[slot, 0] = jnp.sum(mx * mf)
            stats_s[slot, 1] = jnp.sum(me * mf)
            stats_s[slot, 2] = jnp.sum(mn * mf)
            stats_s[slot, 3] = jnp.sum(mx * mx * mf)
            stats_s[slot, 4] = jnp.sum(me * me * mf)
            stats_s[slot, 5] = jnp.sum(mn * mn * mf)
            stats_s[slot, 6] = jnp.sum(mf)

    @pl.when(b >= 1)
    def _norm():
        ps = jax.lax.rem(b + 1, 2)
        cnt = stats_s[ps, 6]
        row0 = k * OH
        keep = maskf_s[ps, pl.ds(row0, OH), :] > 0.0
        for c in range(3):
            s1 = stats_s[ps, c]
            s2 = stats_s[ps, 3 + c]
            mean = s1 / cnt
            var = (s2 - s1 * s1 / cnt) / (cnt - 1.0)
            rstd = jax.lax.rsqrt(var)
            p = pooled_s[ps, c, pl.ds(row0, OH), :]
            out_ref[0, c] = jnp.where(keep, (p - mean) * rstd, 0.0)


@jax.jit
def kernel(x, mask):
    mask = mask.astype(jnp.int32)

    out = pl.pallas_call(
        _fused_kernel,
        grid=(B + 1, NC),
        in_specs=[
            pl.BlockSpec(
                (1, CC, H, W),
                lambda b, k: (jnp.minimum(b, B - 1),
                              jnp.where(b == B, NC - 1, k), 0, 0)),
            pl.BlockSpec(
                (1, H, W),
                lambda b, k: (jnp.minimum(b, B - 1), 0, 0)),
        ],
        out_specs=pl.BlockSpec(
            (1, 3, OH, W),
            lambda b, k: (jnp.maximum(b - 1, 0), 0, k, 0)),
        out_shape=jax.ShapeDtypeStruct((B, 3, H, W), jnp.float32),
        scratch_shapes=[
            pltpu.VMEM((2, 3, H, W), jnp.float32),
            pltpu.VMEM((2, H, W), jnp.float32),
            pltpu.SMEM((2, 8), jnp.float32),
        ],
    )(x, mask)

    return out


# speedup vs baseline: 1.5472x; 1.0919x over previous
import jax
import jax.numpy as jnp
from jax.experimental import pallas as pl
from jax.experimental.pallas import tpu as pltpu

B, C, H, W = 8, 96, 384, 384
CC = 32
NC = C // CC
OH = H // NC


def _fused_kernel(x_ref, m_ref, out_ref, pooled_s, maskf_s, stats_s):
    b = pl.program_id(0)
    k = pl.program_id(1)
    slot = jax.lax.rem(b, 2)

    @pl.when(b < B)
    def _pool():
        xb = x_ref[0]
        cmx = jnp.max(xb, axis=0)
        cmn = jnp.min(xb, axis=0)
        csm = jnp.sum(xb, axis=0)

        @pl.when(k == 0)
        def _():
            pooled_s[slot, 0] = cmx
            pooled_s[slot, 1] = csm
            pooled_s[slot, 2] = cmn
            maskf_s[slot] = (m_ref[0] == 1).astype(jnp.float32)

        @pl.when(k > 0)
        def _():
            pooled_s[slot, 0] = jnp.maximum(pooled_s[slot, 0], cmx)
            pooled_s[slot, 1] = pooled_s[slot, 1] + csm
            pooled_s[slot, 2] = jnp.minimum(pooled_s[slot, 2], cmn)

        @pl.when(k == NC - 1)
        def _():
            me = pooled_s[slot, 1] * (1.0 / C)
            pooled_s[slot, 1] = me
            mf = maskf_s[slot]
            mx = pooled_s[slot, 0]
            mn = pooled_s[slot, 2]
            stats_s[slot, 0] = jnp.sum(mx * mf)
            stats_s[slot, 1] = jnp.sum(me * mf)
            stats_s[slot, 2] = jnp.sum(mn * mf)
            stats_s[slot, 3] = jnp.sum(mx * mx * mf)
            stats_s[slot, 4] = jnp.sum(me * me * mf)
            stats_s[slot, 5] = jnp.sum(mn * mn * mf)
            stats_s[slot, 6] = jnp.sum(mf)

    @pl.when(b >= 1)
    def _norm():
        ps = jax.lax.rem(b + 1, 2)
        cnt = stats_s[ps, 6]
        row0 = k * OH
        keep = maskf_s[ps, pl.ds(row0, OH), :] > 0.0
        for c in range(3):
            s1 = stats_s[ps, c]
            s2 = stats_s[ps, 3 + c]
            mean = s1 / cnt
            var = (s2 - s1 * s1 / cnt) / (cnt - 1.0)
            rstd = jax.lax.rsqrt(var)
            p = pooled_s[ps, c, pl.ds(row0, OH), :]
            out_ref[0, c] = jnp.where(keep, (p - mean) * rstd, 0.0)


@jax.jit
def kernel(x, mask):
    mask = mask.astype(jnp.int32)

    out = pl.pallas_call(
        _fused_kernel,
        grid=(B + 1, NC),
        in_specs=[
            pl.BlockSpec(
                (1, CC, H, W),
                lambda b, k: (jnp.minimum(b, B - 1),
                              jnp.where(b == B, NC - 1, k), 0, 0)),
            pl.BlockSpec(
                (1, H, W),
                lambda b, k: (jnp.minimum(b, B - 1), 0, 0)),
        ],
        out_specs=pl.BlockSpec(
            (1, 3, OH, W),
            lambda b, k: (jnp.maximum(b - 1, 0), 0, k, 0)),
        out_shape=jax.ShapeDtypeStruct((B, 3, H, W), jnp.float32),
        scratch_shapes=[
            pltpu.VMEM((2, 3, H, W), jnp.float32),
            pltpu.VMEM((2, H, W), jnp.float32),
            pltpu.SMEM((2, 8), jnp.float32),
        ],
    )(x, mask)

    return out
